# baseline (speedup 1.0000x reference)
import functools

import jax
import jax.numpy as jnp
from jax import lax
from jax.experimental import pallas as pl
from jax.experimental.pallas import tpu as pltpu
from jax.experimental.pallas import tpu_sc as plsc

NUM_ROWS = 65536
NUM_COLS = 65536
NNZ = 4194304

NW = 32
EPW = NNZ // NW
EPW_SHIFT = 17
CH = 2048
NCHUNK = EPW // CH
G = CH // 16
RPW = NUM_ROWS // NW
PTR_TILE = RPW + 128
PTR_PAD = NUM_ROWS + 128

_mesh = plsc.VectorSubcoreMesh(core_axis_name="c", subcore_axis_name="s")


def _wid():
    return lax.axis_index("s") * 2 + lax.axis_index("c")


@functools.partial(
    pl.kernel,
    mesh=_mesh,
    compiler_params=pltpu.CompilerParams(needs_layout_passes=False),
    out_type=[
        jax.ShapeDtypeStruct((NNZ,), jnp.float32),
        jax.ShapeDtypeStruct((NW * 16,), jnp.float32),
    ],
    scratch_types=[
        pltpu.VMEM((NUM_COLS,), jnp.float32),
        pltpu.VMEM((CH,), jnp.float32),
        pltpu.VMEM((2 * CH,), jnp.int32),
        pltpu.VMEM((CH,), jnp.float32),
        pltpu.VMEM((2 * CH,), jnp.int32),
        pltpu.VMEM((CH,), jnp.float32),
        pltpu.VMEM((CH,), jnp.float32),
        pltpu.VMEM((16,), jnp.float32),
        pltpu.SemaphoreType.DMA,
        pltpu.SemaphoreType.DMA,
        pltpu.SemaphoreType.DMA,
        pltpu.SemaphoreType.DMA,
    ],
)
def _phase1(values_hbm, cols_hbm, x_hbm, p_hbm, tot_hbm,
            x_v, vals_a, cols_a, vals_b, cols_b, out_a, out_b, stage_v,
            sem_ia, sem_ib, sem_oa, sem_ob):
    wid = _wid()
    base = wid * jnp.int32(EPW)
    lane15 = jnp.full((16, 1), 15, jnp.int32)
    evens = lax.iota(jnp.int32, 16) * jnp.int32(2)
    bcast_dnums = lax.GatherDimensionNumbers(
        offset_dims=(), collapsed_slice_dims=(0,), start_index_map=(0,))

    def start_in(c, vv, cv, sem):
        off = base + c * jnp.int32(CH)
        pltpu.async_copy(values_hbm.at[pl.ds(off, CH)], vv, sem)
        pltpu.async_copy(
            cols_hbm.at[pl.ds(off * jnp.int32(2), 2 * CH)], cv, sem)

    def wait_in(vv, cv, sem):
        pltpu.make_async_copy(values_hbm.at[pl.ds(0, CH)], vv, sem).wait()
        pltpu.make_async_copy(cols_hbm.at[pl.ds(0, 2 * CH)], cv, sem).wait()

    def wait_out(ov, sem):
        pltpu.make_async_copy(ov, p_hbm.at[pl.ds(0, CH)], sem).wait()

    start_in(jnp.int32(0), vals_a, cols_a, sem_ia)
    start_in(jnp.int32(1), vals_b, cols_b, sem_ib)
    pltpu.sync_copy(x_hbm, x_v)

    def compute(vv, cv, ov, cin):
        @plsc.parallel_loop(jnp.int32(0), jnp.int32(G), step=jnp.int32(1), unroll=8, carry=cin)
        def group_body(g, cv16):
            gg = g * jnp.int32(16)
            cols16 = plsc.load_gather(cv, [evens + g * jnp.int32(32)])
            vals16 = vv[pl.ds(gg, 16)]
            prod = plsc.load_gather(x_v, [cols16]) * vals16
            pc = plsc.cumsum(prod)
            ov[pl.ds(gg, 16)] = pc + cv16
            last = lax.gather(
                pc, lane15, bcast_dnums, slice_sizes=(1,),
                mode=lax.GatherScatterMode.PROMISE_IN_BOUNDS)
            return cv16 + last
        return group_body

    @pl.loop(jnp.int32(0), jnp.int32(NCHUNK), step=jnp.int32(2),
             init_carry=jnp.zeros((16,), jnp.float32))
    def chunk_pair(c, carry_v):
        @pl.when(c > jnp.int32(0))
        def _():
            wait_out(out_a, sem_oa)
        wait_in(vals_a, cols_a, sem_ia)
        carry_v = compute(vals_a, cols_a, out_a, carry_v)

        @pl.when(c + jnp.int32(2) < jnp.int32(NCHUNK))
        def _():
            start_in(c + jnp.int32(2), vals_a, cols_a, sem_ia)
        pltpu.async_copy(out_a, p_hbm.at[pl.ds(base + c * jnp.int32(CH), CH)],
                         sem_oa)

        @pl.when(c > jnp.int32(0))
        def _():
            wait_out(out_b, sem_ob)
        wait_in(vals_b, cols_b, sem_ib)
        carry_v = compute(vals_b, cols_b, out_b, carry_v)

        @pl.when(c + jnp.int32(3) < jnp.int32(NCHUNK))
        def _():
            start_in(c + jnp.int32(3), vals_b, cols_b, sem_ib)
        pltpu.async_copy(out_b,
                         p_hbm.at[pl.ds(base + (c + jnp.int32(1)) * jnp.int32(CH),
                                        CH)], sem_ob)
        return carry_v

    wait_out(out_a, sem_oa)
    wait_out(out_b, sem_ob)
    stage_v[...] = chunk_pair
    pltpu.sync_copy(stage_v, tot_hbm.at[pl.ds(wid * jnp.int32(16), 16)])


@functools.partial(
    pl.kernel,
    mesh=_mesh,
    compiler_params=pltpu.CompilerParams(needs_layout_passes=False),
    out_type=jax.ShapeDtypeStruct((NUM_ROWS,), jnp.float32),
    scratch_types=[
        pltpu.VMEM((PTR_TILE,), jnp.int32),
        pltpu.VMEM((PTR_TILE,), jnp.int32),
        pltpu.VMEM((PTR_TILE,), jnp.float32),
        pltpu.VMEM((NW * 16,), jnp.float32),
        pltpu.VMEM((NW,), jnp.float32),
        pltpu.VMEM((RPW,), jnp.float32),
        pltpu.SemaphoreType.DMA,
    ],
)
def _phase2(ptr_hbm, p_hbm, tot_hbm, y_hbm,
            ptr_v, pm1_v, pv_v, tot_v, c_v, y_v, sem):
    wid = _wid()
    rbase = wid * jnp.int32(RPW)
    pltpu.sync_copy(ptr_hbm.at[pl.ds(rbase, PTR_TILE)], ptr_v)
    pltpu.sync_copy(tot_hbm, tot_v)

    idx0 = lax.iota(jnp.int32, 16) * jnp.int32(16)
    t0 = plsc.load_gather(tot_v, [idx0])
    t1 = plsc.load_gather(tot_v, [idx0 + jnp.int32(256)])
    c_v[pl.ds(0, 16)] = plsc.cumsum(t0) - t0
    c_v[pl.ds(16, 16)] = plsc.cumsum(t1) - t1 + jnp.sum(t0)

    def pm1_body(k, _):
        kk = k * jnp.int32(16)
        pm1_v[pl.ds(kk, 16)] = jnp.maximum(ptr_v[pl.ds(kk, 16)] - jnp.int32(1),
                                           jnp.int32(0))
        return jnp.int32(0)

    lax.fori_loop(jnp.int32(0), jnp.int32(PTR_TILE // 16), pm1_body,
                  jnp.int32(0))

    def gather_body(b, _):
        pltpu.async_copy(p_hbm.at[pm1_v.at[pl.ds(b * jnp.int32(128), 128)]],
                         pv_v.at[pl.ds(b * jnp.int32(128), 128)], sem)
        return jnp.int32(0)

    lax.fori_loop(jnp.int32(0), jnp.int32(PTR_TILE // 128), gather_body,
                  jnp.int32(0))

    def drain_body(b, _):
        pltpu.make_async_copy(
            p_hbm.at[pm1_v.at[pl.ds(b * jnp.int32(128), 128)]],
            pv_v.at[pl.ds(b * jnp.int32(128), 128)], sem).wait()
        return jnp.int32(0)

    lax.fori_loop(jnp.int32(0), jnp.int32(PTR_TILE // 128), drain_body,
                  jnp.int32(0))

    zero = jnp.zeros((16,), jnp.float32)
    sh = jnp.int32(EPW_SHIFT)

    def row_body(k, _):
        kk = k * jnp.int32(16)
        s16 = ptr_v[pl.ds(kk, 16)]
        e16 = ptr_v[pl.ds(kk + jnp.int32(1), 16)]
        ps = pv_v[pl.ds(kk, 16)]
        pe = pv_v[pl.ds(kk + jnp.int32(1), 16)]
        sm1 = jnp.maximum(s16 - jnp.int32(1), jnp.int32(0))
        em1 = jnp.maximum(e16 - jnp.int32(1), jnp.int32(0))
        cs = plsc.load_gather(c_v, [lax.shift_right_logical(sm1, sh)])
        ce = plsc.load_gather(c_v, [lax.shift_right_logical(em1, sh)])
        es = jnp.where(s16 > jnp.int32(0), ps + cs, zero)
        ee = jnp.where(e16 > jnp.int32(0), pe + ce, zero)
        y_v[pl.ds(kk, 16)] = ee - es
        return jnp.int32(0)

    lax.fori_loop(jnp.int32(0), jnp.int32(RPW // 16), row_body, jnp.int32(0))

    pltpu.sync_copy(y_v, y_hbm.at[pl.ds(rbase, RPW)])


def kernel(values, col_indices, row_ptrs, x):
    values = values.astype(jnp.float32)
    x = x.astype(jnp.float32)
    cols2 = lax.bitcast_convert_type(col_indices, jnp.int32).reshape(-1)
    ptr32 = row_ptrs.astype(jnp.int32)
    ptr_pad = jnp.concatenate(
        [ptr32, jnp.broadcast_to(ptr32[-1], (PTR_PAD - (NUM_ROWS + 1),))])
    p, tot = _phase1(values, cols2, x)
    return _phase2(ptr_pad, p, tot)


# speedup vs baseline: 26.8053x; 26.8053x over previous
import functools

import jax
import jax.numpy as jnp
from jax import lax
from jax.experimental import pallas as pl
from jax.experimental.pallas import tpu as pltpu
from jax.experimental.pallas import tpu_sc as plsc

NUM_ROWS = 65536
NUM_COLS = 65536
NNZ = 4194304

NW = 32
EPW = NNZ // NW
EPW_SHIFT = 17
CH = 2048
NCHUNK = EPW // CH
G = CH // 16
RPW = NUM_ROWS // NW
PTR_TILE = RPW + 128
PTR_PAD = NUM_ROWS + 128

_mesh = plsc.VectorSubcoreMesh(core_axis_name="c", subcore_axis_name="s")


def _wid():
    return lax.axis_index("s") * 2 + lax.axis_index("c")


@functools.partial(
    pl.kernel,
    mesh=_mesh,
    compiler_params=pltpu.CompilerParams(needs_layout_passes=False),
    out_type=[
        jax.ShapeDtypeStruct((NNZ,), jnp.float32),
        jax.ShapeDtypeStruct((NW * 16,), jnp.float32),
    ],
    scratch_types=[
        pltpu.VMEM((NUM_COLS,), jnp.float32),
        pltpu.VMEM((CH,), jnp.float32),
        pltpu.VMEM((CH,), jnp.int32),
        pltpu.VMEM((CH,), jnp.float32),
        pltpu.VMEM((CH,), jnp.int32),
        pltpu.VMEM((CH,), jnp.float32),
        pltpu.VMEM((CH,), jnp.float32),
        pltpu.VMEM((16,), jnp.float32),
        pltpu.SemaphoreType.DMA,
        pltpu.SemaphoreType.DMA,
        pltpu.SemaphoreType.DMA,
        pltpu.SemaphoreType.DMA,
    ],
)
def _phase1(values_hbm, cols_hbm, x_hbm, p_hbm, tot_hbm,
            x_v, vals_a, cols_a, vals_b, cols_b, out_a, out_b, stage_v,
            sem_ia, sem_ib, sem_oa, sem_ob):
    wid = _wid()
    base = wid * jnp.int32(EPW)
    lane15 = jnp.full((16, 1), 15, jnp.int32)
    bcast_dnums = lax.GatherDimensionNumbers(
        offset_dims=(), collapsed_slice_dims=(0,), start_index_map=(0,))

    def start_in(c, vv, cv, sem):
        off = base + c * jnp.int32(CH)
        pltpu.async_copy(values_hbm.at[pl.ds(off, CH)], vv, sem)
        pltpu.async_copy(cols_hbm.at[pl.ds(off, CH)], cv, sem)

    def wait_in(vv, cv, sem):
        pltpu.make_async_copy(values_hbm.at[pl.ds(0, CH)], vv, sem).wait()
        pltpu.make_async_copy(cols_hbm.at[pl.ds(0, CH)], cv, sem).wait()

    def wait_out(ov, sem):
        pltpu.make_async_copy(ov, p_hbm.at[pl.ds(0, CH)], sem).wait()

    start_in(jnp.int32(0), vals_a, cols_a, sem_ia)
    start_in(jnp.int32(1), vals_b, cols_b, sem_ib)
    pltpu.sync_copy(x_hbm, x_v)

    def compute(vv, cv, ov, cin):
        @plsc.parallel_loop(jnp.int32(0), jnp.int32(G), step=jnp.int32(1), unroll=8, carry=cin)
        def group_body(g, cv16):
            gg = g * jnp.int32(16)
            cols16 = cv[pl.ds(gg, 16)]
            vals16 = vv[pl.ds(gg, 16)]
            prod = plsc.load_gather(x_v, [cols16]) * vals16
            pc = plsc.cumsum(prod)
            ov[pl.ds(gg, 16)] = pc + cv16
            last = lax.gather(
                pc, lane15, bcast_dnums, slice_sizes=(1,),
                mode=lax.GatherScatterMode.PROMISE_IN_BOUNDS)
            return cv16 + last
        return group_body

    @pl.loop(jnp.int32(0), jnp.int32(NCHUNK), step=jnp.int32(2),
             init_carry=jnp.zeros((16,), jnp.float32))
    def chunk_pair(c, carry_v):
        @pl.when(c > jnp.int32(0))
        def _():
            wait_out(out_a, sem_oa)
        wait_in(vals_a, cols_a, sem_ia)
        carry_v = compute(vals_a, cols_a, out_a, carry_v)

        @pl.when(c + jnp.int32(2) < jnp.int32(NCHUNK))
        def _():
            start_in(c + jnp.int32(2), vals_a, cols_a, sem_ia)
        pltpu.async_copy(out_a, p_hbm.at[pl.ds(base + c * jnp.int32(CH), CH)],
                         sem_oa)

        @pl.when(c > jnp.int32(0))
        def _():
            wait_out(out_b, sem_ob)
        wait_in(vals_b, cols_b, sem_ib)
        carry_v = compute(vals_b, cols_b, out_b, carry_v)

        @pl.when(c + jnp.int32(3) < jnp.int32(NCHUNK))
        def _():
            start_in(c + jnp.int32(3), vals_b, cols_b, sem_ib)
        pltpu.async_copy(out_b,
                         p_hbm.at[pl.ds(base + (c + jnp.int32(1)) * jnp.int32(CH),
                                        CH)], sem_ob)
        return carry_v

    wait_out(out_a, sem_oa)
    wait_out(out_b, sem_ob)
    stage_v[...] = chunk_pair
    pltpu.sync_copy(stage_v, tot_hbm.at[pl.ds(wid * jnp.int32(16), 16)])


@functools.partial(
    pl.kernel,
    mesh=_mesh,
    compiler_params=pltpu.CompilerParams(needs_layout_passes=False),
    out_type=jax.ShapeDtypeStruct((NUM_ROWS,), jnp.float32),
    scratch_types=[
        pltpu.VMEM((PTR_TILE,), jnp.int32),
        pltpu.VMEM((PTR_TILE,), jnp.int32),
        pltpu.VMEM((PTR_TILE,), jnp.float32),
        pltpu.VMEM((NW * 16,), jnp.float32),
        pltpu.VMEM((NW,), jnp.float32),
        pltpu.VMEM((RPW,), jnp.float32),
        pltpu.SemaphoreType.DMA,
    ],
)
def _phase2(ptr_hbm, p_hbm, tot_hbm, y_hbm,
            ptr_v, pm1_v, pv_v, tot_v, c_v, y_v, sem):
    wid = _wid()
    rbase = wid * jnp.int32(RPW)
    pltpu.sync_copy(ptr_hbm.at[pl.ds(rbase, PTR_TILE)], ptr_v)
    pltpu.sync_copy(tot_hbm, tot_v)

    idx0 = lax.iota(jnp.int32, 16) * jnp.int32(16)
    t0 = plsc.load_gather(tot_v, [idx0])
    t1 = plsc.load_gather(tot_v, [idx0 + jnp.int32(256)])
    c_v[pl.ds(0, 16)] = plsc.cumsum(t0) - t0
    c_v[pl.ds(16, 16)] = plsc.cumsum(t1) - t1 + jnp.sum(t0)

    def pm1_body(k, _):
        kk = k * jnp.int32(16)
        pm1_v[pl.ds(kk, 16)] = jnp.maximum(ptr_v[pl.ds(kk, 16)] - jnp.int32(1),
                                           jnp.int32(0))
        return jnp.int32(0)

    lax.fori_loop(jnp.int32(0), jnp.int32(PTR_TILE // 16), pm1_body,
                  jnp.int32(0))

    def gather_body(b, _):
        pltpu.async_copy(p_hbm.at[pm1_v.at[pl.ds(b * jnp.int32(128), 128)]],
                         pv_v.at[pl.ds(b * jnp.int32(128), 128)], sem)
        return jnp.int32(0)

    lax.fori_loop(jnp.int32(0), jnp.int32(PTR_TILE // 128), gather_body,
                  jnp.int32(0))

    def drain_body(b, _):
        pltpu.make_async_copy(
            p_hbm.at[pm1_v.at[pl.ds(b * jnp.int32(128), 128)]],
            pv_v.at[pl.ds(b * jnp.int32(128), 128)], sem).wait()
        return jnp.int32(0)

    lax.fori_loop(jnp.int32(0), jnp.int32(PTR_TILE // 128), drain_body,
                  jnp.int32(0))

    zero = jnp.zeros((16,), jnp.float32)
    sh = jnp.int32(EPW_SHIFT)

    def row_body(k, _):
        kk = k * jnp.int32(16)
        s16 = ptr_v[pl.ds(kk, 16)]
        e16 = ptr_v[pl.ds(kk + jnp.int32(1), 16)]
        ps = pv_v[pl.ds(kk, 16)]
        pe = pv_v[pl.ds(kk + jnp.int32(1), 16)]
        sm1 = jnp.maximum(s16 - jnp.int32(1), jnp.int32(0))
        em1 = jnp.maximum(e16 - jnp.int32(1), jnp.int32(0))
        cs = plsc.load_gather(c_v, [lax.shift_right_logical(sm1, sh)])
        ce = plsc.load_gather(c_v, [lax.shift_right_logical(em1, sh)])
        es = jnp.where(s16 > jnp.int32(0), ps + cs, zero)
        ee = jnp.where(e16 > jnp.int32(0), pe + ce, zero)
        y_v[pl.ds(kk, 16)] = ee - es
        return jnp.int32(0)

    lax.fori_loop(jnp.int32(0), jnp.int32(RPW // 16), row_body, jnp.int32(0))

    pltpu.sync_copy(y_v, y_hbm.at[pl.ds(rbase, RPW)])


def kernel(values, col_indices, row_ptrs, x):
    values = values.astype(jnp.float32)
    x = x.astype(jnp.float32)
    cols32 = col_indices.astype(jnp.int32)
    ptr32 = row_ptrs.astype(jnp.int32)
    ptr_pad = jnp.concatenate(
        [ptr32, jnp.broadcast_to(ptr32[-1], (PTR_PAD - (NUM_ROWS + 1),))])
    p, tot = _phase1(values, cols32, x)
    return _phase2(ptr_pad, p, tot)


# speedup vs baseline: 28.3371x; 1.0571x over previous
import functools

import jax
import jax.numpy as jnp
from jax import lax
from jax.experimental import pallas as pl
from jax.experimental.pallas import tpu as pltpu
from jax.experimental.pallas import tpu_sc as plsc

NUM_ROWS = 65536
NUM_COLS = 65536
NNZ = 4194304

NW = 32
EPW = NNZ // NW
EPW_SHIFT = 17
CH = 2048
NCHUNK = EPW // CH
G = CH // 16
RPW = NUM_ROWS // NW
PTR_TILE = RPW + 128
PTR_PAD = NUM_ROWS + 128

_mesh = plsc.VectorSubcoreMesh(core_axis_name="c", subcore_axis_name="s")


def _wid():
    return lax.axis_index("s") * 2 + lax.axis_index("c")


@functools.partial(
    pl.kernel,
    mesh=_mesh,
    compiler_params=pltpu.CompilerParams(needs_layout_passes=False),
    out_type=[
        jax.ShapeDtypeStruct((NNZ,), jnp.float32),
        jax.ShapeDtypeStruct((NW * 16,), jnp.float32),
    ],
    scratch_types=[
        pltpu.VMEM((NUM_COLS,), jnp.float32),
        pltpu.VMEM((CH,), jnp.float32),
        pltpu.VMEM((CH,), jnp.uint32),
        pltpu.VMEM((CH,), jnp.float32),
        pltpu.VMEM((CH,), jnp.uint32),
        pltpu.VMEM((CH,), jnp.float32),
        pltpu.VMEM((CH,), jnp.float32),
        pltpu.VMEM((16,), jnp.float32),
        pltpu.SemaphoreType.DMA,
        pltpu.SemaphoreType.DMA,
        pltpu.SemaphoreType.DMA,
        pltpu.SemaphoreType.DMA,
    ],
)
def _phase1(values_hbm, cols_hbm, x_hbm, p_hbm, tot_hbm,
            x_v, vals_a, cols_a, vals_b, cols_b, out_a, out_b, stage_v,
            sem_ia, sem_ib, sem_oa, sem_ob):
    wid = _wid()
    base = wid * jnp.int32(EPW)
    lane15 = jnp.full((16, 1), 15, jnp.int32)
    bcast_dnums = lax.GatherDimensionNumbers(
        offset_dims=(), collapsed_slice_dims=(0,), start_index_map=(0,))

    def start_in(c, vv, cv, sem):
        off = base + c * jnp.int32(CH)
        pltpu.async_copy(values_hbm.at[pl.ds(off, CH)], vv, sem)
        pltpu.async_copy(cols_hbm.at[pl.ds(off, CH)], cv, sem)

    def wait_in(vv, cv, sem):
        pltpu.make_async_copy(values_hbm.at[pl.ds(0, CH)], vv, sem).wait()
        pltpu.make_async_copy(cols_hbm.at[pl.ds(0, CH)], cv, sem).wait()

    def wait_out(ov, sem):
        pltpu.make_async_copy(ov, p_hbm.at[pl.ds(0, CH)], sem).wait()

    start_in(jnp.int32(0), vals_a, cols_a, sem_ia)
    start_in(jnp.int32(1), vals_b, cols_b, sem_ib)
    pltpu.sync_copy(x_hbm, x_v)

    def compute(vv, cv, ov, cin):
        @plsc.parallel_loop(jnp.int32(0), jnp.int32(G), step=jnp.int32(1), unroll=8, carry=cin)
        def group_body(g, cv16):
            gg = g * jnp.int32(16)
            cols16 = plsc.bitcast(cv[pl.ds(gg, 16)], jnp.int32)
            vals16 = vv[pl.ds(gg, 16)]
            prod = plsc.load_gather(x_v, [cols16]) * vals16
            pc = plsc.cumsum(prod)
            ov[pl.ds(gg, 16)] = pc + cv16
            last = lax.gather(
                pc, lane15, bcast_dnums, slice_sizes=(1,),
                mode=lax.GatherScatterMode.PROMISE_IN_BOUNDS)
            return cv16 + last
        return group_body

    @pl.loop(jnp.int32(0), jnp.int32(NCHUNK), step=jnp.int32(2),
             init_carry=jnp.zeros((16,), jnp.float32))
    def chunk_pair(c, carry_v):
        @pl.when(c > jnp.int32(0))
        def _():
            wait_out(out_a, sem_oa)
        wait_in(vals_a, cols_a, sem_ia)
        carry_v = compute(vals_a, cols_a, out_a, carry_v)

        @pl.when(c + jnp.int32(2) < jnp.int32(NCHUNK))
        def _():
            start_in(c + jnp.int32(2), vals_a, cols_a, sem_ia)
        pltpu.async_copy(out_a, p_hbm.at[pl.ds(base + c * jnp.int32(CH), CH)],
                         sem_oa)

        @pl.when(c > jnp.int32(0))
        def _():
            wait_out(out_b, sem_ob)
        wait_in(vals_b, cols_b, sem_ib)
        carry_v = compute(vals_b, cols_b, out_b, carry_v)

        @pl.when(c + jnp.int32(3) < jnp.int32(NCHUNK))
        def _():
            start_in(c + jnp.int32(3), vals_b, cols_b, sem_ib)
        pltpu.async_copy(out_b,
                         p_hbm.at[pl.ds(base + (c + jnp.int32(1)) * jnp.int32(CH),
                                        CH)], sem_ob)
        return carry_v

    wait_out(out_a, sem_oa)
    wait_out(out_b, sem_ob)
    stage_v[...] = chunk_pair
    pltpu.sync_copy(stage_v, tot_hbm.at[pl.ds(wid * jnp.int32(16), 16)])


@functools.partial(
    pl.kernel,
    mesh=_mesh,
    compiler_params=pltpu.CompilerParams(needs_layout_passes=False),
    out_type=jax.ShapeDtypeStruct((NUM_ROWS,), jnp.float32),
    scratch_types=[
        pltpu.VMEM((PTR_TILE,), jnp.uint32),
        pltpu.VMEM((PTR_TILE,), jnp.int32),
        pltpu.VMEM((PTR_TILE,), jnp.float32),
        pltpu.VMEM((NW * 16,), jnp.float32),
        pltpu.VMEM((NW,), jnp.float32),
        pltpu.VMEM((RPW,), jnp.float32),
        pltpu.SemaphoreType.DMA,
    ],
)
def _phase2(ptr_hbm, p_hbm, tot_hbm, y_hbm,
            ptr_v, pm1_v, pv_v, tot_v, c_v, y_v, sem):
    wid = _wid()
    rbase = wid * jnp.int32(RPW)
    pltpu.sync_copy(ptr_hbm.at[pl.ds(rbase, PTR_TILE)], ptr_v)
    pltpu.sync_copy(tot_hbm, tot_v)

    idx0 = lax.iota(jnp.int32, 16) * jnp.int32(16)
    t0 = plsc.load_gather(tot_v, [idx0])
    t1 = plsc.load_gather(tot_v, [idx0 + jnp.int32(256)])
    c_v[pl.ds(0, 16)] = plsc.cumsum(t0) - t0
    c_v[pl.ds(16, 16)] = plsc.cumsum(t1) - t1 + jnp.sum(t0)

    def pm1_body(k, _):
        kk = k * jnp.int32(16)
        p16 = plsc.bitcast(ptr_v[pl.ds(kk, 16)], jnp.int32)
        pm1_v[pl.ds(kk, 16)] = jnp.maximum(p16 - jnp.int32(1), jnp.int32(0))
        return jnp.int32(0)

    lax.fori_loop(jnp.int32(0), jnp.int32(PTR_TILE // 16), pm1_body,
                  jnp.int32(0))

    def gather_body(b, _):
        pltpu.async_copy(p_hbm.at[pm1_v.at[pl.ds(b * jnp.int32(128), 128)]],
                         pv_v.at[pl.ds(b * jnp.int32(128), 128)], sem)
        return jnp.int32(0)

    lax.fori_loop(jnp.int32(0), jnp.int32(PTR_TILE // 128), gather_body,
                  jnp.int32(0))

    def drain_body(b, _):
        pltpu.make_async_copy(
            p_hbm.at[pm1_v.at[pl.ds(b * jnp.int32(128), 128)]],
            pv_v.at[pl.ds(b * jnp.int32(128), 128)], sem).wait()
        return jnp.int32(0)

    lax.fori_loop(jnp.int32(0), jnp.int32(PTR_TILE // 128), drain_body,
                  jnp.int32(0))

    zero = jnp.zeros((16,), jnp.float32)
    sh = jnp.int32(EPW_SHIFT)

    def row_body(k, _):
        kk = k * jnp.int32(16)
        s16 = plsc.bitcast(ptr_v[pl.ds(kk, 16)], jnp.int32)
        e16 = plsc.bitcast(ptr_v[pl.ds(kk + jnp.int32(1), 16)], jnp.int32)
        ps = pv_v[pl.ds(kk, 16)]
        pe = pv_v[pl.ds(kk + jnp.int32(1), 16)]
        sm1 = jnp.maximum(s16 - jnp.int32(1), jnp.int32(0))
        em1 = jnp.maximum(e16 - jnp.int32(1), jnp.int32(0))
        cs = plsc.load_gather(c_v, [lax.shift_right_logical(sm1, sh)])
        ce = plsc.load_gather(c_v, [lax.shift_right_logical(em1, sh)])
        es = jnp.where(s16 > jnp.int32(0), ps + cs, zero)
        ee = jnp.where(e16 > jnp.int32(0), pe + ce, zero)
        y_v[pl.ds(kk, 16)] = ee - es
        return jnp.int32(0)

    lax.fori_loop(jnp.int32(0), jnp.int32(RPW // 16), row_body, jnp.int32(0))

    pltpu.sync_copy(y_v, y_hbm.at[pl.ds(rbase, RPW)])


def kernel(values, col_indices, row_ptrs, x):
    values = values.astype(jnp.float32)
    x = x.astype(jnp.float32)
    cols32 = col_indices.astype(jnp.uint32)
    ptr32 = row_ptrs.astype(jnp.uint32)
    ptr_pad = jnp.concatenate(
        [ptr32, jnp.broadcast_to(ptr32[-1], (PTR_PAD - (NUM_ROWS + 1),))])
    p, tot = _phase1(values, cols32, x)
    return _phase2(ptr_pad, p, tot)


# speedup vs baseline: 29.9296x; 1.0562x over previous
import functools

import jax
import jax.numpy as jnp
from jax import lax
from jax.experimental import pallas as pl
from jax.experimental.pallas import tpu as pltpu
from jax.experimental.pallas import tpu_sc as plsc

NUM_ROWS = 65536
NUM_COLS = 65536
NNZ = 4194304

NW = 32
EPW = NNZ // NW
EPW_SHIFT = 17
CH = 4096
NCHUNK = EPW // CH
G = CH // 16
RPW = NUM_ROWS // NW
PTR_TILE = RPW + 128
PTR_PAD = NUM_ROWS + 128

_mesh = plsc.VectorSubcoreMesh(core_axis_name="c", subcore_axis_name="s")


def _wid():
    return lax.axis_index("s") * 2 + lax.axis_index("c")


@functools.partial(
    pl.kernel,
    mesh=_mesh,
    compiler_params=pltpu.CompilerParams(needs_layout_passes=False),
    out_type=[
        jax.ShapeDtypeStruct((NNZ,), jnp.float32),
        jax.ShapeDtypeStruct((NW * 16,), jnp.float32),
    ],
    scratch_types=[
        pltpu.VMEM((NUM_COLS,), jnp.float32),
        pltpu.VMEM((CH,), jnp.float32),
        pltpu.VMEM((CH,), jnp.uint32),
        pltpu.VMEM((CH,), jnp.float32),
        pltpu.VMEM((CH,), jnp.uint32),
        pltpu.VMEM((CH,), jnp.float32),
        pltpu.VMEM((CH,), jnp.float32),
        pltpu.VMEM((16,), jnp.float32),
        pltpu.SemaphoreType.DMA,
        pltpu.SemaphoreType.DMA,
        pltpu.SemaphoreType.DMA,
        pltpu.SemaphoreType.DMA,
    ],
)
def _phase1(values_hbm, cols_hbm, x_hbm, p_hbm, tot_hbm,
            x_v, vals_a, cols_a, vals_b, cols_b, out_a, out_b, stage_v,
            sem_ia, sem_ib, sem_oa, sem_ob):
    wid = _wid()
    base = wid * jnp.int32(EPW)
    lane15 = jnp.full((16, 1), 15, jnp.int32)
    bcast_dnums = lax.GatherDimensionNumbers(
        offset_dims=(), collapsed_slice_dims=(0,), start_index_map=(0,))

    def start_in(c, vv, cv, sem):
        off = base + c * jnp.int32(CH)
        pltpu.async_copy(values_hbm.at[pl.ds(off, CH)], vv, sem)
        pltpu.async_copy(cols_hbm.at[pl.ds(off, CH)], cv, sem)

    def wait_in(vv, cv, sem):
        pltpu.make_async_copy(values_hbm.at[pl.ds(0, CH)], vv, sem).wait()
        pltpu.make_async_copy(cols_hbm.at[pl.ds(0, CH)], cv, sem).wait()

    def wait_out(ov, sem):
        pltpu.make_async_copy(ov, p_hbm.at[pl.ds(0, CH)], sem).wait()

    start_in(jnp.int32(0), vals_a, cols_a, sem_ia)
    start_in(jnp.int32(1), vals_b, cols_b, sem_ib)
    pltpu.sync_copy(x_hbm, x_v)

    def compute(vv, cv, ov, cin):
        @plsc.parallel_loop(jnp.int32(0), jnp.int32(G), step=jnp.int32(1), unroll=8, carry=cin)
        def group_body(g, cv16):
            gg = g * jnp.int32(16)
            cols16 = plsc.bitcast(cv[pl.ds(gg, 16)], jnp.int32)
            vals16 = vv[pl.ds(gg, 16)]
            prod = plsc.load_gather(x_v, [cols16]) * vals16
            pc = plsc.cumsum(prod)
            ov[pl.ds(gg, 16)] = pc + cv16
            last = lax.gather(
                pc, lane15, bcast_dnums, slice_sizes=(1,),
                mode=lax.GatherScatterMode.PROMISE_IN_BOUNDS)
            return cv16 + last
        return group_body

    @pl.loop(jnp.int32(0), jnp.int32(NCHUNK), step=jnp.int32(2),
             init_carry=jnp.zeros((16,), jnp.float32))
    def chunk_pair(c, carry_v):
        @pl.when(c > jnp.int32(0))
        def _():
            wait_out(out_a, sem_oa)
        wait_in(vals_a, cols_a, sem_ia)
        carry_v = compute(vals_a, cols_a, out_a, carry_v)

        @pl.when(c + jnp.int32(2) < jnp.int32(NCHUNK))
        def _():
            start_in(c + jnp.int32(2), vals_a, cols_a, sem_ia)
        pltpu.async_copy(out_a, p_hbm.at[pl.ds(base + c * jnp.int32(CH), CH)],
                         sem_oa)

        @pl.when(c > jnp.int32(0))
        def _():
            wait_out(out_b, sem_ob)
        wait_in(vals_b, cols_b, sem_ib)
        carry_v = compute(vals_b, cols_b, out_b, carry_v)

        @pl.when(c + jnp.int32(3) < jnp.int32(NCHUNK))
        def _():
            start_in(c + jnp.int32(3), vals_b, cols_b, sem_ib)
        pltpu.async_copy(out_b,
                         p_hbm.at[pl.ds(base + (c + jnp.int32(1)) * jnp.int32(CH),
                                        CH)], sem_ob)
        return carry_v

    wait_out(out_a, sem_oa)
    wait_out(out_b, sem_ob)
    stage_v[...] = chunk_pair
    pltpu.sync_copy(stage_v, tot_hbm.at[pl.ds(wid * jnp.int32(16), 16)])


@functools.partial(
    pl.kernel,
    mesh=_mesh,
    compiler_params=pltpu.CompilerParams(needs_layout_passes=False),
    out_type=jax.ShapeDtypeStruct((NUM_ROWS,), jnp.float32),
    scratch_types=[
        pltpu.VMEM((PTR_TILE,), jnp.uint32),
        pltpu.VMEM((PTR_TILE,), jnp.int32),
        pltpu.VMEM((PTR_TILE,), jnp.float32),
        pltpu.VMEM((NW * 16,), jnp.float32),
        pltpu.VMEM((NW,), jnp.float32),
        pltpu.VMEM((RPW,), jnp.float32),
        pltpu.SemaphoreType.DMA,
    ],
)
def _phase2(ptr_hbm, p_hbm, tot_hbm, y_hbm,
            ptr_v, pm1_v, pv_v, tot_v, c_v, y_v, sem):
    wid = _wid()
    rbase = wid * jnp.int32(RPW)
    pltpu.sync_copy(ptr_hbm.at[pl.ds(rbase, PTR_TILE)], ptr_v)
    pltpu.sync_copy(tot_hbm, tot_v)

    idx0 = lax.iota(jnp.int32, 16) * jnp.int32(16)
    t0 = plsc.load_gather(tot_v, [idx0])
    t1 = plsc.load_gather(tot_v, [idx0 + jnp.int32(256)])
    c_v[pl.ds(0, 16)] = plsc.cumsum(t0) - t0
    c_v[pl.ds(16, 16)] = plsc.cumsum(t1) - t1 + jnp.sum(t0)

    def pm1_body(k, _):
        kk = k * jnp.int32(16)
        p16 = plsc.bitcast(ptr_v[pl.ds(kk, 16)], jnp.int32)
        pm1_v[pl.ds(kk, 16)] = jnp.maximum(p16 - jnp.int32(1), jnp.int32(0))
        return jnp.int32(0)

    lax.fori_loop(jnp.int32(0), jnp.int32(PTR_TILE // 16), pm1_body,
                  jnp.int32(0))

    def gather_body(b, _):
        pltpu.async_copy(p_hbm.at[pm1_v.at[pl.ds(b * jnp.int32(128), 128)]],
                         pv_v.at[pl.ds(b * jnp.int32(128), 128)], sem)
        return jnp.int32(0)

    lax.fori_loop(jnp.int32(0), jnp.int32(PTR_TILE // 128), gather_body,
                  jnp.int32(0))

    def drain_body(b, _):
        pltpu.make_async_copy(
            p_hbm.at[pm1_v.at[pl.ds(b * jnp.int32(128), 128)]],
            pv_v.at[pl.ds(b * jnp.int32(128), 128)], sem).wait()
        return jnp.int32(0)

    lax.fori_loop(jnp.int32(0), jnp.int32(PTR_TILE // 128), drain_body,
                  jnp.int32(0))

    zero = jnp.zeros((16,), jnp.float32)
    sh = jnp.int32(EPW_SHIFT)

    def row_body(k, _):
        kk = k * jnp.int32(16)
        s16 = plsc.bitcast(ptr_v[pl.ds(kk, 16)], jnp.int32)
        e16 = plsc.bitcast(ptr_v[pl.ds(kk + jnp.int32(1), 16)], jnp.int32)
        ps = pv_v[pl.ds(kk, 16)]
        pe = pv_v[pl.ds(kk + jnp.int32(1), 16)]
        sm1 = jnp.maximum(s16 - jnp.int32(1), jnp.int32(0))
        em1 = jnp.maximum(e16 - jnp.int32(1), jnp.int32(0))
        cs = plsc.load_gather(c_v, [lax.shift_right_logical(sm1, sh)])
        ce = plsc.load_gather(c_v, [lax.shift_right_logical(em1, sh)])
        es = jnp.where(s16 > jnp.int32(0), ps + cs, zero)
        ee = jnp.where(e16 > jnp.int32(0), pe + ce, zero)
        y_v[pl.ds(kk, 16)] = ee - es
        return jnp.int32(0)

    lax.fori_loop(jnp.int32(0), jnp.int32(RPW // 16), row_body, jnp.int32(0))

    pltpu.sync_copy(y_v, y_hbm.at[pl.ds(rbase, RPW)])


def kernel(values, col_indices, row_ptrs, x):
    values = values.astype(jnp.float32)
    x = x.astype(jnp.float32)
    cols32 = col_indices.astype(jnp.uint32)
    ptr32 = row_ptrs.astype(jnp.uint32)
    ptr_pad = jnp.concatenate(
        [ptr32, jnp.broadcast_to(ptr32[-1], (PTR_PAD - (NUM_ROWS + 1),))])
    p, tot = _phase1(values, cols32, x)
    return _phase2(ptr_pad, p, tot)


# speedup vs baseline: 30.7801x; 1.0284x over previous
import functools

import jax
import jax.numpy as jnp
from jax import lax
from jax.experimental import pallas as pl
from jax.experimental.pallas import tpu as pltpu
from jax.experimental.pallas import tpu_sc as plsc

NUM_ROWS = 65536
NUM_COLS = 65536
NNZ = 4194304

NW = 32
EPW = NNZ // NW
EPW_SHIFT = 17
CH = 8192
NCHUNK = EPW // CH
G = CH // 16
RPW = NUM_ROWS // NW
PTR_TILE = RPW + 128
PTR_PAD = NUM_ROWS + 128

_mesh = plsc.VectorSubcoreMesh(core_axis_name="c", subcore_axis_name="s")


def _wid():
    return lax.axis_index("s") * 2 + lax.axis_index("c")


@functools.partial(
    pl.kernel,
    mesh=_mesh,
    compiler_params=pltpu.CompilerParams(needs_layout_passes=False),
    out_type=[
        jax.ShapeDtypeStruct((NNZ,), jnp.float32),
        jax.ShapeDtypeStruct((NW * 16,), jnp.float32),
    ],
    scratch_types=[
        pltpu.VMEM((NUM_COLS,), jnp.float32),
        pltpu.VMEM((CH,), jnp.float32),
        pltpu.VMEM((CH,), jnp.uint32),
        pltpu.VMEM((CH,), jnp.float32),
        pltpu.VMEM((CH,), jnp.uint32),
        pltpu.VMEM((CH,), jnp.float32),
        pltpu.VMEM((CH,), jnp.float32),
        pltpu.VMEM((16,), jnp.float32),
        pltpu.SemaphoreType.DMA,
        pltpu.SemaphoreType.DMA,
        pltpu.SemaphoreType.DMA,
        pltpu.SemaphoreType.DMA,
    ],
)
def _phase1(values_hbm, cols_hbm, x_hbm, p_hbm, tot_hbm,
            x_v, vals_a, cols_a, vals_b, cols_b, out_a, out_b, stage_v,
            sem_ia, sem_ib, sem_oa, sem_ob):
    wid = _wid()
    base = wid * jnp.int32(EPW)
    lane15 = jnp.full((16, 1), 15, jnp.int32)
    bcast_dnums = lax.GatherDimensionNumbers(
        offset_dims=(), collapsed_slice_dims=(0,), start_index_map=(0,))

    def start_in(c, vv, cv, sem):
        off = base + c * jnp.int32(CH)
        pltpu.async_copy(values_hbm.at[pl.ds(off, CH)], vv, sem)
        pltpu.async_copy(cols_hbm.at[pl.ds(off, CH)], cv, sem)

    def wait_in(vv, cv, sem):
        pltpu.make_async_copy(values_hbm.at[pl.ds(0, CH)], vv, sem).wait()
        pltpu.make_async_copy(cols_hbm.at[pl.ds(0, CH)], cv, sem).wait()

    def wait_out(ov, sem):
        pltpu.make_async_copy(ov, p_hbm.at[pl.ds(0, CH)], sem).wait()

    start_in(jnp.int32(0), vals_a, cols_a, sem_ia)
    start_in(jnp.int32(1), vals_b, cols_b, sem_ib)
    pltpu.sync_copy(x_hbm, x_v)

    def compute(vv, cv, ov, cin):
        @plsc.parallel_loop(jnp.int32(0), jnp.int32(G), step=jnp.int32(1), unroll=8, carry=cin)
        def group_body(g, cv16):
            gg = g * jnp.int32(16)
            cols16 = plsc.bitcast(cv[pl.ds(gg, 16)], jnp.int32)
            vals16 = vv[pl.ds(gg, 16)]
            prod = plsc.load_gather(x_v, [cols16]) * vals16
            pc = plsc.cumsum(prod)
            ov[pl.ds(gg, 16)] = pc + cv16
            last = lax.gather(
                pc, lane15, bcast_dnums, slice_sizes=(1,),
                mode=lax.GatherScatterMode.PROMISE_IN_BOUNDS)
            return cv16 + last
        return group_body

    @pl.loop(jnp.int32(0), jnp.int32(NCHUNK), step=jnp.int32(2),
             init_carry=jnp.zeros((16,), jnp.float32))
    def chunk_pair(c, carry_v):
        @pl.when(c > jnp.int32(0))
        def _():
            wait_out(out_a, sem_oa)
        wait_in(vals_a, cols_a, sem_ia)
        carry_v = compute(vals_a, cols_a, out_a, carry_v)

        @pl.when(c + jnp.int32(2) < jnp.int32(NCHUNK))
        def _():
            start_in(c + jnp.int32(2), vals_a, cols_a, sem_ia)
        pltpu.async_copy(out_a, p_hbm.at[pl.ds(base + c * jnp.int32(CH), CH)],
                         sem_oa)

        @pl.when(c > jnp.int32(0))
        def _():
            wait_out(out_b, sem_ob)
        wait_in(vals_b, cols_b, sem_ib)
        carry_v = compute(vals_b, cols_b, out_b, carry_v)

        @pl.when(c + jnp.int32(3) < jnp.int32(NCHUNK))
        def _():
            start_in(c + jnp.int32(3), vals_b, cols_b, sem_ib)
        pltpu.async_copy(out_b,
                         p_hbm.at[pl.ds(base + (c + jnp.int32(1)) * jnp.int32(CH),
                                        CH)], sem_ob)
        return carry_v

    wait_out(out_a, sem_oa)
    wait_out(out_b, sem_ob)
    stage_v[...] = chunk_pair
    pltpu.sync_copy(stage_v, tot_hbm.at[pl.ds(wid * jnp.int32(16), 16)])


@functools.partial(
    pl.kernel,
    mesh=_mesh,
    compiler_params=pltpu.CompilerParams(needs_layout_passes=False),
    out_type=jax.ShapeDtypeStruct((NUM_ROWS,), jnp.float32),
    scratch_types=[
        pltpu.VMEM((PTR_TILE,), jnp.uint32),
        pltpu.VMEM((PTR_TILE,), jnp.int32),
        pltpu.VMEM((PTR_TILE,), jnp.float32),
        pltpu.VMEM((NW * 16,), jnp.float32),
        pltpu.VMEM((NW,), jnp.float32),
        pltpu.VMEM((RPW,), jnp.float32),
        pltpu.SemaphoreType.DMA,
    ],
)
def _phase2(ptr_hbm, p_hbm, tot_hbm, y_hbm,
            ptr_v, pm1_v, pv_v, tot_v, c_v, y_v, sem):
    wid = _wid()
    rbase = wid * jnp.int32(RPW)
    pltpu.sync_copy(ptr_hbm.at[pl.ds(rbase, PTR_TILE)], ptr_v)
    pltpu.sync_copy(tot_hbm, tot_v)

    idx0 = lax.iota(jnp.int32, 16) * jnp.int32(16)
    t0 = plsc.load_gather(tot_v, [idx0])
    t1 = plsc.load_gather(tot_v, [idx0 + jnp.int32(256)])
    c_v[pl.ds(0, 16)] = plsc.cumsum(t0) - t0
    c_v[pl.ds(16, 16)] = plsc.cumsum(t1) - t1 + jnp.sum(t0)

    def pm1_body(k, _):
        kk = k * jnp.int32(16)
        p16 = plsc.bitcast(ptr_v[pl.ds(kk, 16)], jnp.int32)
        pm1_v[pl.ds(kk, 16)] = jnp.maximum(p16 - jnp.int32(1), jnp.int32(0))
        return jnp.int32(0)

    lax.fori_loop(jnp.int32(0), jnp.int32(PTR_TILE // 16), pm1_body,
                  jnp.int32(0))

    def gather_body(b, _):
        pltpu.async_copy(p_hbm.at[pm1_v.at[pl.ds(b * jnp.int32(128), 128)]],
                         pv_v.at[pl.ds(b * jnp.int32(128), 128)], sem)
        return jnp.int32(0)

    lax.fori_loop(jnp.int32(0), jnp.int32(PTR_TILE // 128), gather_body,
                  jnp.int32(0))

    def drain_body(b, _):
        pltpu.make_async_copy(
            p_hbm.at[pm1_v.at[pl.ds(b * jnp.int32(128), 128)]],
            pv_v.at[pl.ds(b * jnp.int32(128), 128)], sem).wait()
        return jnp.int32(0)

    lax.fori_loop(jnp.int32(0), jnp.int32(PTR_TILE // 128), drain_body,
                  jnp.int32(0))

    zero = jnp.zeros((16,), jnp.float32)
    sh = jnp.int32(EPW_SHIFT)

    def row_body(k, _):
        kk = k * jnp.int32(16)
        s16 = plsc.bitcast(ptr_v[pl.ds(kk, 16)], jnp.int32)
        e16 = plsc.bitcast(ptr_v[pl.ds(kk + jnp.int32(1), 16)], jnp.int32)
        ps = pv_v[pl.ds(kk, 16)]
        pe = pv_v[pl.ds(kk + jnp.int32(1), 16)]
        sm1 = jnp.maximum(s16 - jnp.int32(1), jnp.int32(0))
        em1 = jnp.maximum(e16 - jnp.int32(1), jnp.int32(0))
        cs = plsc.load_gather(c_v, [lax.shift_right_logical(sm1, sh)])
        ce = plsc.load_gather(c_v, [lax.shift_right_logical(em1, sh)])
        es = jnp.where(s16 > jnp.int32(0), ps + cs, zero)
        ee = jnp.where(e16 > jnp.int32(0), pe + ce, zero)
        y_v[pl.ds(kk, 16)] = ee - es
        return jnp.int32(0)

    lax.fori_loop(jnp.int32(0), jnp.int32(RPW // 16), row_body, jnp.int32(0))

    pltpu.sync_copy(y_v, y_hbm.at[pl.ds(rbase, RPW)])


def kernel(values, col_indices, row_ptrs, x):
    values = values.astype(jnp.float32)
    x = x.astype(jnp.float32)
    cols32 = col_indices.astype(jnp.uint32)
    ptr32 = row_ptrs.astype(jnp.uint32)
    ptr_pad = jnp.concatenate(
        [ptr32, jnp.broadcast_to(ptr32[-1], (PTR_PAD - (NUM_ROWS + 1),))])
    p, tot = _phase1(values, cols32, x)
    return _phase2(ptr_pad, p, tot)
